# baseline (speedup 1.0000x reference)
import functools

import jax
import jax.numpy as jnp
from jax import lax
from jax.experimental import pallas as pl
from jax.experimental.pallas import tpu as pltpu
from jax.experimental.pallas import tpu_sc as plsc

_HIDDEN = 64
_CHUNK = 128


@functools.lru_cache(maxsize=None)
def _build(total, hidden):
    info = plsc.get_sparse_core_info()
    nc, ns = info.num_cores, info.num_subcores
    nw = nc * ns
    per_w = total // nw
    nchunk = per_w // _CHUNK
    assert per_w * nw == total and nchunk * _CHUNK == per_w

    mesh = plsc.VectorSubcoreMesh(core_axis_name="c", subcore_axis_name="s")

    @functools.partial(
        pl.kernel,
        mesh=mesh,
        out_type=jax.ShapeDtypeStruct((total, hidden), jnp.float32),
        compiler_params=pltpu.CompilerParams(use_tc_tiling_on_sc=False),
        scratch_types=[
            pltpu.VMEM((nchunk, _CHUNK), jnp.int32),
            pltpu.VMEM((_CHUNK, hidden), jnp.float32),
            pltpu.SemaphoreType.DMA,
        ],
    )
    def gather_k(table_hbm, idx_hbm, out_hbm, idx_v, rows_v, sem):
        wid = lax.axis_index("s") * nc + lax.axis_index("c")
        base = wid * per_w
        pltpu.sync_copy(idx_hbm.at[wid], idx_v)

        def body(c, carry):
            pltpu.async_copy(table_hbm.at[idx_v.at[c]], rows_v, sem).wait()
            pltpu.sync_copy(rows_v, out_hbm.at[pl.ds(base + c * _CHUNK, _CHUNK)])
            return carry

        lax.fori_loop(0, nchunk, body, 0)

    def run(embedding, idx_flat):
        idx3 = idx_flat.reshape(nw, nchunk, _CHUNK)
        return gather_k(embedding, idx3)

    return run


def kernel(inputs, embedding):
    b, h = inputs.shape
    total = b * h
    hidden = embedding.shape[1]
    idx_flat = inputs.reshape(total).astype(jnp.int32)
    out = _build(total, hidden)(embedding, idx_flat)
    return out.reshape(b, h, hidden)


# speedup vs baseline: 1.0505x; 1.0505x over previous
import functools

import jax
import jax.numpy as jnp
from jax import lax
from jax.experimental import pallas as pl
from jax.experimental.pallas import tpu as pltpu
from jax.experimental.pallas import tpu_sc as plsc

_HIDDEN = 64
_CHUNK = 128
_NBUF = 4


@functools.lru_cache(maxsize=None)
def _build(total, hidden):
    info = plsc.get_sparse_core_info()
    nc, ns = info.num_cores, info.num_subcores
    nw = nc * ns
    per_w = total // nw
    nchunk = per_w // _CHUNK
    assert per_w * nw == total and nchunk * _CHUNK == per_w

    mesh = plsc.VectorSubcoreMesh(core_axis_name="c", subcore_axis_name="s")

    @functools.partial(
        pl.kernel,
        mesh=mesh,
        out_type=jax.ShapeDtypeStruct((total, hidden), jnp.float32),
        compiler_params=pltpu.CompilerParams(use_tc_tiling_on_sc=False),
        scratch_types=[
            pltpu.VMEM((nchunk, _CHUNK), jnp.int32),
            pltpu.VMEM((_NBUF, _CHUNK, hidden), jnp.float32),
            pltpu.SemaphoreType.DMA((_NBUF,)),
        ],
    )
    def gather_k(table_hbm, idx_hbm, out_hbm, idx_v, rows_v, gsem):
        wid = lax.axis_index("s") * nc + lax.axis_index("c")
        base = wid * per_w
        pltpu.sync_copy(idx_hbm.at[wid], idx_v)

        for j in range(_NBUF):
            pltpu.async_copy(table_hbm.at[idx_v.at[j]], rows_v.at[j], gsem.at[j])

        def body(c, carry):
            p = lax.rem(c, _NBUF)
            pltpu.make_async_copy(table_hbm.at[idx_v.at[p]], rows_v.at[p], gsem.at[p]).wait()
            pltpu.sync_copy(rows_v.at[p], out_hbm.at[pl.ds(base + c * _CHUNK, _CHUNK)])
            pltpu.async_copy(table_hbm.at[idx_v.at[c + _NBUF]], rows_v.at[p], gsem.at[p])
            return carry

        lax.fori_loop(0, nchunk - _NBUF, body, 0)

        for j in range(nchunk - _NBUF, nchunk):
            p = j % _NBUF
            pltpu.make_async_copy(table_hbm.at[idx_v.at[p]], rows_v.at[p], gsem.at[p]).wait()
            pltpu.sync_copy(rows_v.at[p], out_hbm.at[pl.ds(base + j * _CHUNK, _CHUNK)])

    def run(embedding, idx_flat):
        idx3 = idx_flat.reshape(nw, nchunk, _CHUNK)
        return gather_k(embedding, idx3)

    return run


def kernel(inputs, embedding):
    b, h = inputs.shape
    total = b * h
    hidden = embedding.shape[1]
    idx_flat = inputs.reshape(total).astype(jnp.int32)
    out = _build(total, hidden)(embedding, idx_flat)
    return out.reshape(b, h, hidden)
